# baseline (speedup 1.0000x reference)
import functools

import jax
import jax.numpy as jnp
from jax import lax
from jax.experimental import pallas as pl
from jax.experimental.pallas import tpu as pltpu
from jax.experimental.pallas import tpu_sc as plsc

N = 10000
E = 320000
D = 128
B = 64
OUT = 128
T0 = 120
T1 = 64

NW = 32
EW = E // NW
CH = 128
NCH = 80
EWP = NCH * CH
HD = D // 2
EW2 = E // 16
NCH2 = 160
EWP2 = NCH2 * CH
NCHN = 79
NPAD = NCHN * CH
TRASH = N
MROWS = 10240
STRIPE = MROWS // 16

_mesh = plsc.VectorSubcoreMesh(core_axis_name="c", subcore_axis_name="s")
_f32 = jnp.float32


def _zero_rows(buf):
    zv = jnp.zeros((16,), _f32)

    def body(r, carry):
        for k in range(D // 16):
            buf[r, pl.ds(16 * k, 16)] = zv
        return carry

    lax.fori_loop(0, CH, body, 0)


def _sc_prep_body(src_hbm, dst_hbm, degs_hbm,
                  dout_sp, din_sp, zb, onesb, srcv, dstv, sem):
    cid = lax.axis_index("c")
    sid = lax.axis_index("s")
    wid = cid * 16 + sid

    zv = jnp.zeros((16,), _f32)
    ov = jnp.ones((16,), _f32)
    for k in range(STRIPE // 16):
        zb[pl.ds(16 * k, 16)] = zv
    for k in range(CH // 16):
        onesb[pl.ds(16 * k, 16)] = ov

    pltpu.sync_copy(zb, dout_sp.at[pl.ds(STRIPE * sid, STRIPE)])
    pltpu.sync_copy(zb, din_sp.at[pl.ds(STRIPE * sid, STRIPE)])
    pltpu.sync_copy(src_hbm.at[wid], srcv)
    pltpu.sync_copy(dst_hbm.at[wid], dstv)
    plsc.subcore_barrier()

    def deg_body(t, carry):
        for b in range(8):
            j = 8 * t + b
            pltpu.async_copy(onesb, dout_sp.at[srcv.at[j]], sem, add=True)
            pltpu.async_copy(onesb, din_sp.at[dstv.at[j]], sem, add=True)
        for _ in range(16):
            pltpu.make_async_copy(onesb, dout_sp.at[srcv.at[0]], sem).wait()
        return carry

    lax.fori_loop(0, NCH // 8, deg_body, 0)

    plsc.subcore_barrier()
    sl = pl.ds(STRIPE * sid, STRIPE)
    pltpu.sync_copy(dout_sp.at[sl], degs_hbm.at[cid, 0, sl])
    pltpu.sync_copy(din_sp.at[sl], degs_hbm.at[cid, 1, sl])


_sc_prep = functools.partial(
    pl.kernel,
    out_type=jax.ShapeDtypeStruct((2, 2, MROWS), _f32),
    mesh=_mesh,
    scratch_types=[
        pltpu.VMEM_SHARED((MROWS,), _f32),
        pltpu.VMEM_SHARED((MROWS,), _f32),
        pltpu.VMEM((STRIPE,), _f32),
        pltpu.VMEM((CH,), _f32),
        pltpu.VMEM((NCH, CH), jnp.int32),
        pltpu.VMEM((NCH, CH), jnp.int32),
        pltpu.SemaphoreType.DMA,
    ],
)(_sc_prep_body)


def _sc_msg_body(h_hbm, src_hbm, dst_hbm, out_hbm,
                 m_sp, si0, si1, di0, di1, r0, r1,
                 gs0, gs1, is0, is1):
    cid = lax.axis_index("c")
    sid = lax.axis_index("s")
    wid = cid * 16 + sid
    rows = (r0, r1)
    gsem = (gs0, gs1)
    sidx = (si0, si1)
    didx = (di0, di1)
    isem = (is0, is1)

    _zero_rows(r0)
    for t in range(STRIPE // CH):
        pltpu.sync_copy(r0, m_sp.at[pl.ds(STRIPE * sid + CH * t, CH)])
    plsc.subcore_barrier()

    pltpu.sync_copy(src_hbm.at[wid, 0], si0)
    pltpu.sync_copy(dst_hbm.at[wid, 0], di0)
    pltpu.async_copy(src_hbm.at[wid, 1], si1, is1)
    pltpu.async_copy(dst_hbm.at[wid, 1], di1, is1)
    pltpu.async_copy(h_hbm.at[si0], r0, gs0)

    def step(j, b, b2):
        @pl.when(j + 1 < NCH)
        def _():
            pltpu.make_async_copy(src_hbm.at[wid, 0], sidx[b2],
                                  isem[b2]).wait()
            pltpu.make_async_copy(src_hbm.at[wid, 0], didx[b2],
                                  isem[b2]).wait()
            pltpu.async_copy(h_hbm.at[sidx[b2]], rows[b2], gsem[b2])

        pltpu.make_async_copy(h_hbm.at[pl.ds(0, CH)], rows[b],
                              gsem[b]).wait()
        pltpu.sync_copy(rows[b], m_sp.at[didx[b]], add=True)

        @pl.when(j + 2 < NCH)
        def _():
            pltpu.async_copy(src_hbm.at[wid, j + 2], sidx[b], isem[b])
            pltpu.async_copy(dst_hbm.at[wid, j + 2], didx[b], isem[b])

    def outer(t, carry):
        j0 = 2 * t
        step(j0, 0, 1)
        step(j0 + 1, 1, 0)
        return carry

    lax.fori_loop(0, NCH // 2, outer, 0)

    plsc.subcore_barrier()
    sl = pl.ds(STRIPE * sid, STRIPE)
    pltpu.sync_copy(m_sp.at[sl], out_hbm.at[cid, sl])


_sc_msg = functools.partial(
    pl.kernel,
    out_type=jax.ShapeDtypeStruct((2, MROWS, D), _f32),
    mesh=_mesh,
    scratch_types=[
        pltpu.VMEM_SHARED((MROWS, D), _f32),
        pltpu.VMEM((CH,), jnp.int32),
        pltpu.VMEM((CH,), jnp.int32),
        pltpu.VMEM((CH,), jnp.int32),
        pltpu.VMEM((CH,), jnp.int32),
        pltpu.VMEM((CH, D), _f32),
        pltpu.VMEM((CH, D), _f32),
        pltpu.SemaphoreType.DMA,
        pltpu.SemaphoreType.DMA,
        pltpu.SemaphoreType.DMA,
        pltpu.SemaphoreType.DMA,
    ],
)(_sc_msg_body)


def _tc_prep_body(dego_ref, degi_ref, f0_ref, f1_ref, e0_ref, e1_ref,
                  h1_ref, nin_ref, nout_ref):
    do_ = dego_ref[0] + dego_ref[1]
    di = degi_ref[0] + degi_ref[1]
    no = lax.rsqrt(jnp.maximum(do_, 1.0))
    ni = lax.rsqrt(jnp.maximum(di, 1.0))
    nout_ref[...] = no
    nin_ref[...] = ni
    oh0 = (f0_ref[...] ==
           lax.broadcasted_iota(jnp.int32, (CH, T0), 1)).astype(_f32)
    oh1 = (f1_ref[...] ==
           lax.broadcasted_iota(jnp.int32, (CH, T1), 1)).astype(_f32)
    hv = (jnp.dot(oh0, e0_ref[...], preferred_element_type=_f32)
          + jnp.dot(oh1, e1_ref[...], preferred_element_type=_f32))
    h1_ref[...] = hv * no


def _tc_dense_body(p_ref, nin_ref, nout_ref, w_ref, b_ref, h_ref):
    m = (p_ref[0] + p_ref[1]) * nin_ref[...]
    h = jnp.dot(m, w_ref[...], preferred_element_type=_f32) + b_ref[...]
    h_ref[...] = jnp.maximum(h, 0.0) * nout_ref[...]


def _tc_final_body(p_ref, nin_ref, gid_ref, w_ref, b_ref, wout_ref, bout_ref,
                   out_ref, acc, cnt):
    i = pl.program_id(0)
    m = (p_ref[0] + p_ref[1]) * nin_ref[...]
    h = jnp.maximum(
        jnp.dot(m, w_ref[...], preferred_element_type=_f32) + b_ref[...], 0.0)
    gid = gid_ref[...][:, 0]
    oh = (lax.broadcasted_iota(jnp.int32, (B, CH), 0) == gid[None, :]).astype(_f32)
    part = jnp.dot(oh, h, preferred_element_type=_f32)
    pcnt = jnp.sum(oh, axis=1, keepdims=True)

    @pl.when(i == 0)
    def _():
        acc[...] = part
        cnt[...] = pcnt

    @pl.when(i > 0)
    def _():
        acc[...] += part
        cnt[...] += pcnt

    @pl.when(i == NCHN - 1)
    def _():
        hg = acc[...] / jnp.maximum(cnt[...], 1.0)
        out_ref[...] = (
            jnp.dot(hg, wout_ref[...], preferred_element_type=_f32)
            + bout_ref[...])


def _tc_prep(dego, degi, f0, f1, e0, e1):
    return pl.pallas_call(
        _tc_prep_body,
        grid=(NCHN,),
        in_specs=[
            pl.BlockSpec((2, CH, 1), lambda i: (0, i, 0)),
            pl.BlockSpec((2, CH, 1), lambda i: (0, i, 0)),
            pl.BlockSpec((CH, 1), lambda i: (i, 0)),
            pl.BlockSpec((CH, 1), lambda i: (i, 0)),
            pl.BlockSpec((T0, D), lambda i: (0, 0)),
            pl.BlockSpec((T1, D), lambda i: (0, 0)),
        ],
        out_specs=[
            pl.BlockSpec((CH, D), lambda i: (i, 0)),
            pl.BlockSpec((CH, 1), lambda i: (i, 0)),
            pl.BlockSpec((CH, 1), lambda i: (i, 0)),
        ],
        out_shape=[
            jax.ShapeDtypeStruct((NPAD, D), _f32),
            jax.ShapeDtypeStruct((NPAD, 1), _f32),
            jax.ShapeDtypeStruct((NPAD, 1), _f32),
        ],
    )(dego, degi, f0, f1, e0, e1)


def _tc_dense(p, nin, nout, w, b):
    return pl.pallas_call(
        _tc_dense_body,
        grid=(NCHN,),
        in_specs=[
            pl.BlockSpec((2, CH, D), lambda i: (0, i, 0)),
            pl.BlockSpec((CH, 1), lambda i: (i, 0)),
            pl.BlockSpec((CH, 1), lambda i: (i, 0)),
            pl.BlockSpec((D, D), lambda i: (0, 0)),
            pl.BlockSpec((1, D), lambda i: (0, 0)),
        ],
        out_specs=pl.BlockSpec((CH, D), lambda i: (i, 0)),
        out_shape=jax.ShapeDtypeStruct((NPAD, D), _f32),
    )(p, nin, nout, w, b)


def _tc_final(p, nin, gid, w, b, wout, bout):
    return pl.pallas_call(
        _tc_final_body,
        grid=(NCHN,),
        in_specs=[
            pl.BlockSpec((2, CH, D), lambda i: (0, i, 0)),
            pl.BlockSpec((CH, 1), lambda i: (i, 0)),
            pl.BlockSpec((CH, 1), lambda i: (i, 0)),
            pl.BlockSpec((D, D), lambda i: (0, 0)),
            pl.BlockSpec((1, D), lambda i: (0, 0)),
            pl.BlockSpec((D, OUT), lambda i: (0, 0)),
            pl.BlockSpec((1, OUT), lambda i: (0, 0)),
        ],
        out_specs=pl.BlockSpec((B, OUT), lambda i: (0, 0)),
        out_shape=jax.ShapeDtypeStruct((B, OUT), _f32),
        scratch_shapes=[
            pltpu.VMEM((B, D), _f32),
            pltpu.VMEM((B, 1), _f32),
        ],
    )(p, nin, gid, w, b, wout, bout)


def kernel(feats0, feats1, edge_index, graph_ids,
           emb0, emb1, W0, b0, W1, b1, W2, b2, Wout, bout):
    src = edge_index[0].reshape(NW, EW)
    dst = edge_index[1].reshape(NW, EW)
    srcp = jnp.pad(src, ((0, 0), (0, EWP - EW)),
                   constant_values=TRASH).reshape(NW, NCH, CH)
    dstp = jnp.pad(dst, ((0, 0), (0, EWP - EW)),
                   constant_values=TRASH).reshape(NW, NCH, CH)
    f0p = jnp.pad(feats0, (0, NPAD - N)).reshape(NPAD, 1)
    f1p = jnp.pad(feats1, (0, NPAD - N)).reshape(NPAD, 1)
    gidp = jnp.pad(graph_ids, (0, NPAD - N),
                   constant_values=B).reshape(NPAD, 1)

    degs = _sc_prep(srcp, dstp)
    dego = degs[:, 0, :NPAD].reshape(2, NPAD, 1)
    degi = degs[:, 1, :NPAD].reshape(2, NPAD, 1)

    h, nin, nout = _tc_prep(dego, degi, f0p, f1p, emb0, emb1)
    for w, b_ in ((W0, b0), (W1, b1)):
        p = _sc_msg(h, srcp, dstp)
        h = _tc_dense(p, nin, nout, w, b_.reshape(1, D))
    p = _sc_msg(h, srcp, dstp)
    return _tc_final(p, nin, gidp, W2, b2.reshape(1, D),
                     Wout, bout.reshape(1, OUT))
